# Initial kernel scaffold; baseline (speedup 1.0000x reference)
import jax
import jax.numpy as jnp
from jax.experimental import pallas as pl


def kernel(codes, codebook):
    raise NotImplementedError("write your pallas kernel here")


# speedup vs baseline: 57.8246x; 57.8246x over previous
import jax
import jax.numpy as jnp
from jax.experimental import pallas as pl
from jax.experimental.pallas import tpu as pltpu

_K = 512
_D = 256


def _vq_body(x_ref, cb_ref, out_ref):
    x = x_ref[...]
    cb = cb_ref[...]
    s = jnp.sum(cb, axis=0, keepdims=True)
    q = jnp.sum(cb * cb, axis=0, keepdims=True)
    dist2 = jnp.float32(_K) * (x * x) - 2.0 * x * s + q
    dist = jnp.sqrt(jnp.maximum(dist2, 0.0))
    m = jnp.min(dist, axis=1, keepdims=True)
    iota_d = jax.lax.broadcasted_iota(jnp.int32, dist.shape, 1)
    idx = jnp.min(jnp.where(dist == m, iota_d, _D), axis=1)
    oh = (iota_d == idx[:, None]).astype(jnp.float32)
    gathered = jax.lax.dot_general(
        oh, cb[:_D, :], (((1,), (0,)), ((), ())),
        preferred_element_type=jnp.float32,
        precision=jax.lax.Precision.HIGHEST)
    out_ref[...] = x + (gathered - x)


def kernel(codes, codebook):
    b, t, d = codes.shape
    x = codes.reshape(b * t, d)
    out = pl.pallas_call(
        _vq_body,
        out_shape=jax.ShapeDtypeStruct((b * t, d), jnp.float32),
    )(x, codebook)
    return out.reshape(b, t, d)
